# baseline (speedup 1.0000x reference)
import functools
import math

import jax
import jax.numpy as jnp
from jax import lax
from jax.experimental import pallas as pl
from jax.experimental.pallas import tpu as pltpu
from jax.experimental.pallas import tpu_sc as plsc

NRG = 20
RMAX = 10.0
EF = 4
NH = 16


def _node_mlp_kernel(h_ref, w1_ref, b1_ref, g_ref, bb_ref, w2_ref, b2_ref, o_ref):
    y = jnp.dot(h_ref[...], w1_ref[...], preferred_element_type=jnp.float32)
    y = y + b1_ref[...]
    mu = jnp.mean(y, axis=-1, keepdims=True)
    var = jnp.mean((y - mu) ** 2, axis=-1, keepdims=True)
    y = (y - mu) * jax.lax.rsqrt(var + 1e-5) * g_ref[...] + bb_ref[...]
    y = jax.nn.relu(y)
    o_ref[...] = jnp.dot(y, w2_ref[...], preferred_element_type=jnp.float32) + b2_ref[...]


def _q_mlp(h, w1, b1, g, b, w2, b2):
    n, hid = h.shape
    tn = 400 if n % 400 == 0 else 128
    grid = pl.cdiv(n, tn)
    full = lambda r, c: pl.BlockSpec((r, c), lambda i: (0, 0))
    return pl.pallas_call(
        _node_mlp_kernel,
        grid=(grid,),
        in_specs=[
            pl.BlockSpec((tn, hid), lambda i: (i, 0)),
            full(hid, hid), full(1, hid), full(1, hid), full(1, hid),
            full(hid, hid), full(1, hid),
        ],
        out_specs=pl.BlockSpec((tn, hid), lambda i: (i, 0)),
        out_shape=jax.ShapeDtypeStruct((n, hid), jnp.float32),
    )(h, w1, b1.reshape(1, -1), g.reshape(1, -1), b.reshape(1, -1),
      w2, b2.reshape(1, -1))


def _edge_kernel(gdst_ref, gsrc_ref, xd_ref, xs_ref, et_ref, ew_ref, dstm_ref,
                 w1h_ref, w1et_ref, w1rf_ref, b1_ref, g_ref, bb_ref,
                 w2_ref, b2_ref, mh_ref, mht_ref, om_ref, oe_ref, *, hid):
    coeff = -0.5 / (RMAX / (NRG - 1)) ** 2
    hi = gdst_ref[:, 0:hid].astype(jnp.bfloat16)
    qd = gdst_ref[:, hid:2 * hid]
    hj = gsrc_ref[...].astype(jnp.bfloat16)
    diff = xd_ref[...] - xs_ref[...]
    d = jnp.sqrt(jnp.sum(diff * diff, axis=-1, keepdims=True) + 1e-12)
    offs = jax.lax.broadcasted_iota(jnp.int32, (1, NRG), 1).astype(jnp.float32) * (RMAX / (NRG - 1))
    rf = jnp.exp(coeff * (d - offs) ** 2)
    et = et_ref[...]
    hcat = jnp.concatenate([hi, hj], axis=1)
    y = jnp.dot(hcat, w1h_ref[...], preferred_element_type=jnp.float32)
    y = y + jnp.dot(et.astype(jnp.bfloat16), w1et_ref[...],
                    preferred_element_type=jnp.float32)
    for f in range(EF):
        wf = w1rf_ref[f * NRG:(f + 1) * NRG, :]
        y = y + et[:, f:f + 1] * jnp.dot(rf.astype(jnp.bfloat16), wf,
                                         preferred_element_type=jnp.float32)
    y = y + b1_ref[...]
    yk = y[:, :hid]
    yv = y[:, hid:]

    def ln(z):
        mu = jnp.mean(z, axis=-1, keepdims=True)
        var = jnp.mean((z - mu) ** 2, axis=-1, keepdims=True)
        return (z - mu) * jax.lax.rsqrt(var + 1e-5)

    y = jnp.concatenate([ln(yk), ln(yv)], axis=1) * g_ref[...] + bb_ref[...]
    y = jax.nn.relu(y)
    kv = jnp.dot(y.astype(jnp.bfloat16), w2_ref[...],
                 preferred_element_type=jnp.float32) + b2_ref[...]
    k = kv[:, :hid]
    v = kv[:, hid:] * ew_ref[...]
    hd = hid // NH
    qk = (qd.astype(jnp.float32) * k).astype(jnp.bfloat16)
    s = jnp.dot(qk, mh_ref[...], preferred_element_type=jnp.float32)
    ex = jnp.exp(s * (1.0 / math.sqrt(hd)))
    m = jnp.dot(ex.astype(jnp.bfloat16), mht_ref[...],
                preferred_element_type=jnp.float32) * v
    ex8 = jnp.concatenate([ex] * (hid // NH), axis=1)
    slot = jax.lax.broadcasted_iota(jnp.int32, (1, hid), 1) // NH
    oh = (dstm_ref[...].astype(jnp.int32) == slot).astype(jnp.float32)
    om_ref[...] = m
    oe_ref[...] = ex8 * oh


def _edge_pass(gdst, gsrc, xd, xs, edge_type, e_w, dstm8, w1h, w1et, w1rf,
               b1, g, bb, w2, b2, mh, mht, hid):
    e = gdst.shape[0]
    t = 1280 if e % 1280 == 0 else 128
    grid = pl.cdiv(e, t)
    full = lambda r, c: pl.BlockSpec((r, c), lambda i: (0, 0))
    return pl.pallas_call(
        functools.partial(_edge_kernel, hid=hid),
        grid=(grid,),
        in_specs=[
            pl.BlockSpec((t, 2 * hid), lambda i: (i, 0)),
            pl.BlockSpec((t, hid), lambda i: (i, 0)),
            pl.BlockSpec((t, 16), lambda i: (i, 0)),
            pl.BlockSpec((t, 16), lambda i: (i, 0)),
            pl.BlockSpec((t, EF), lambda i: (i, 0)),
            pl.BlockSpec((t, 1), lambda i: (i, 0)),
            pl.BlockSpec((t, 1), lambda i: (i, 0)),
            full(2 * hid, 2 * hid), full(EF, 2 * hid), full(EF * NRG, 2 * hid),
            full(1, 2 * hid), full(1, 2 * hid), full(1, 2 * hid),
            full(2 * hid, 2 * hid), full(1, 2 * hid),
            full(hid, NH), full(NH, hid),
        ],
        out_specs=[pl.BlockSpec((t, hid), lambda i: (i, 0)),
                   pl.BlockSpec((t, hid), lambda i: (i, 0))],
        out_shape=[jax.ShapeDtypeStruct((e, hid), jnp.float32),
                   jax.ShapeDtypeStruct((e, hid), jnp.float32)],
    )(gdst, gsrc, xd, xs, edge_type, e_w, dstm8, w1h, w1et, w1rf, b1, g, bb,
      w2, b2, mh, mht)


def _sc_scatter(m, exs, dst, dst2, n, hid):
    e = m.shape[0]
    nc, ns = 2, 16
    nw = nc * ns
    per_w = e // nw
    cb = 80
    n_chunks = per_w // cb
    npad = ((n + 127) // 128) * 128
    arows = ((npad + npad // 8 + 127) // 128) * 128
    rpt = arows // ns
    zeros = jnp.zeros((arows, hid), jnp.float32)
    mesh = plsc.VectorSubcoreMesh(core_axis_name="c", subcore_axis_name="s")

    @functools.partial(
        pl.kernel, mesh=mesh,
        out_type=jax.ShapeDtypeStruct((nc, arows, hid), jnp.float32),
        scratch_types=[
            pltpu.VMEM((cb,), jnp.int32),
            pltpu.VMEM((cb,), jnp.int32),
            pltpu.VMEM((cb, hid), jnp.float32),
            pltpu.VMEM((cb, hid), jnp.float32),
            pltpu.VMEM_SHARED((arows, hid), jnp.float32),
        ],
    )
    def body(m_hbm, exs_hbm, dst_hbm, dst2_hbm, z_hbm, out_hbm, idx_v, idx2_v,
             rows_v, rows2_v, acc_sh):
        c = lax.axis_index("c")
        s = lax.axis_index("s")
        wid = c * ns + s
        base_e = wid * per_w
        pltpu.sync_copy(z_hbm.at[pl.ds(s * rpt, rpt)], acc_sh.at[pl.ds(s * rpt, rpt)])
        plsc.subcore_barrier()

        def step(i, carry):
            off = base_e + i * cb
            pltpu.sync_copy(dst_hbm.at[pl.ds(off, cb)], idx_v)
            pltpu.sync_copy(dst2_hbm.at[pl.ds(off, cb)], idx2_v)
            pltpu.sync_copy(m_hbm.at[pl.ds(off, cb)], rows_v)
            pltpu.sync_copy(exs_hbm.at[pl.ds(off, cb)], rows2_v)
            pltpu.sync_copy(rows_v, acc_sh.at[idx_v], add=True)
            pltpu.sync_copy(rows2_v, acc_sh.at[idx2_v], add=True)
            return carry

        lax.fori_loop(0, n_chunks, step, 0)
        plsc.subcore_barrier()
        pltpu.sync_copy(acc_sh.at[pl.ds(s * rpt, rpt)],
                        out_hbm.at[c].at[pl.ds(s * rpt, rpt)])

    return body(m, exs, dst, dst2, zeros)


def kernel(x, h, edge_type, edge_index, e_w, gen_flag,
           hq_w1, hq_b1, hq_ln_g, hq_ln_b, hq_w2, hq_b2,
           hk_w1, hk_b1, hk_ln_g, hk_ln_b, hk_w2, hk_b2,
           hv_w1, hv_b1, hv_ln_g, hv_ln_b, hv_w2, hv_b2):
    n, hid = h.shape
    hd = hid // NH
    src = edge_index[0].astype(jnp.int32)
    dst = edge_index[1].astype(jnp.int32)

    rs = EF + EF * NRG
    w1h = jnp.concatenate([
        jnp.concatenate([hk_w1[rs:rs + hid], hv_w1[rs:rs + hid]], axis=1),
        jnp.concatenate([hk_w1[rs + hid:], hv_w1[rs + hid:]], axis=1),
    ], axis=0)
    w1et = jnp.concatenate([hk_w1[0:EF], hv_w1[0:EF]], axis=1)
    w1rf = jnp.concatenate([hk_w1[EF:rs], hv_w1[EF:rs]], axis=1)
    b1 = jnp.concatenate([hk_b1, hv_b1]).reshape(1, -1)
    g = jnp.concatenate([hk_ln_g, hv_ln_g]).reshape(1, -1)
    bb = jnp.concatenate([hk_ln_b, hv_ln_b]).reshape(1, -1)
    zero = jnp.zeros((hid, hid), jnp.float32)
    w2 = jnp.concatenate([
        jnp.concatenate([hk_w2, zero], axis=1),
        jnp.concatenate([zero, hv_w2], axis=1),
    ], axis=0)
    b2 = jnp.concatenate([hk_b2, hv_b2]).reshape(1, -1)
    mh = (jax.lax.broadcasted_iota(jnp.int32, (hid, NH), 0) // hd ==
          jax.lax.broadcasted_iota(jnp.int32, (hid, NH), 1)).astype(jnp.bfloat16)
    mht = mh.T
    w1h = w1h.astype(jnp.bfloat16)
    w1et = w1et.astype(jnp.bfloat16)
    w1rf = w1rf.astype(jnp.bfloat16)
    w2 = w2.astype(jnp.bfloat16)

    q = _q_mlp(h, hq_w1, hq_b1, hq_ln_g, hq_ln_b, hq_w2, hq_b2)

    xpad = jnp.pad(x, ((0, 0), (0, 16 - x.shape[1])))
    tdst = jnp.concatenate([h, q], axis=1)
    tsrc = h
    gdst = tdst[dst]
    gsrc = tsrc[src]
    xd = xpad[dst]
    xs = xpad[src]

    dstm8 = (dst % 8).astype(jnp.float32).reshape(-1, 1)
    m, exs = _edge_pass(gdst, gsrc, xd, xs, edge_type, e_w, dstm8, w1h, w1et,
                        w1rf, b1, g, bb, w2, b2, mh, mht, hid)

    npad = ((n + 127) // 128) * 128
    dst2 = npad + dst // 8
    parts = _sc_scatter(m, exs, dst, dst2, n, hid)
    acc = parts[0] + parts[1]
    num = acc[:n, :]
    den = acc[npad:].reshape(-1, NH)[:n, :]
    out = num / (jnp.repeat(den, hd, axis=1) + 1e-16) + h
    return out


# speedup vs baseline: 1.4001x; 1.4001x over previous
import functools
import math

import jax
import jax.numpy as jnp
from jax import lax
from jax.experimental import pallas as pl
from jax.experimental.pallas import tpu as pltpu
from jax.experimental.pallas import tpu_sc as plsc

NRG = 20
RMAX = 10.0
EF = 4
NH = 16


def _node_mlp_kernel(h_ref, w1_ref, b1_ref, g_ref, bb_ref, w2_ref, b2_ref, o_ref):
    y = jnp.dot(h_ref[...], w1_ref[...], preferred_element_type=jnp.float32)
    y = y + b1_ref[...]
    mu = jnp.mean(y, axis=-1, keepdims=True)
    var = jnp.mean((y - mu) ** 2, axis=-1, keepdims=True)
    y = (y - mu) * jax.lax.rsqrt(var + 1e-5) * g_ref[...] + bb_ref[...]
    y = jax.nn.relu(y)
    o_ref[...] = jnp.dot(y, w2_ref[...], preferred_element_type=jnp.float32) + b2_ref[...]


def _q_mlp(h, w1, b1, g, b, w2, b2):
    n, hid = h.shape
    tn = 400 if n % 400 == 0 else 128
    grid = pl.cdiv(n, tn)
    full = lambda r, c: pl.BlockSpec((r, c), lambda i: (0, 0))
    return pl.pallas_call(
        _node_mlp_kernel,
        grid=(grid,),
        in_specs=[
            pl.BlockSpec((tn, hid), lambda i: (i, 0)),
            full(hid, hid), full(1, hid), full(1, hid), full(1, hid),
            full(hid, hid), full(1, hid),
        ],
        out_specs=pl.BlockSpec((tn, hid), lambda i: (i, 0)),
        out_shape=jax.ShapeDtypeStruct((n, hid), jnp.float32),
    )(h, w1, b1.reshape(1, -1), g.reshape(1, -1), b.reshape(1, -1),
      w2, b2.reshape(1, -1))


def _edge_kernel(gdst_ref, gsrc_ref, et_ref, ew_ref, dstm_ref,
                 w1h_ref, w1et_ref, w1rf_ref, b1_ref, g_ref, bb_ref,
                 w2_ref, b2_ref, mh_ref, mht_ref, om_ref, oe_ref, *, hid):
    coeff = -0.5 / (RMAX / (NRG - 1)) ** 2
    hi = gdst_ref[:, 0:hid].astype(jnp.bfloat16)
    qd = gdst_ref[:, hid:2 * hid]
    hj = gsrc_ref[:, 0:hid].astype(jnp.bfloat16)
    diff = gdst_ref[:, 2 * hid:2 * hid + 16] - gsrc_ref[:, hid:hid + 16]
    d = jnp.sqrt(jnp.sum(diff * diff, axis=-1, keepdims=True) + 1e-12)
    offs = jax.lax.broadcasted_iota(jnp.int32, (1, NRG), 1).astype(jnp.float32) * (RMAX / (NRG - 1))
    rf = jnp.exp(coeff * (d - offs) ** 2)
    et = et_ref[...]
    hcat = jnp.concatenate([hi, hj], axis=1)
    y = jnp.dot(hcat, w1h_ref[...], preferred_element_type=jnp.float32)
    y = y + jnp.dot(et.astype(jnp.bfloat16), w1et_ref[...],
                    preferred_element_type=jnp.float32)
    for f in range(EF):
        wf = w1rf_ref[f * NRG:(f + 1) * NRG, :]
        y = y + et[:, f:f + 1] * jnp.dot(rf.astype(jnp.bfloat16), wf,
                                         preferred_element_type=jnp.float32)
    y = y + b1_ref[...]
    yk = y[:, :hid]
    yv = y[:, hid:]

    def ln(z):
        mu = jnp.mean(z, axis=-1, keepdims=True)
        var = jnp.mean((z - mu) ** 2, axis=-1, keepdims=True)
        return (z - mu) * jax.lax.rsqrt(var + 1e-5)

    y = jnp.concatenate([ln(yk), ln(yv)], axis=1) * g_ref[...] + bb_ref[...]
    y = jax.nn.relu(y)
    kv = jnp.dot(y.astype(jnp.bfloat16), w2_ref[...],
                 preferred_element_type=jnp.float32) + b2_ref[...]
    k = kv[:, :hid]
    v = kv[:, hid:] * ew_ref[...]
    hd = hid // NH
    qk = (qd.astype(jnp.float32) * k).astype(jnp.bfloat16)
    s = jnp.dot(qk, mh_ref[...], preferred_element_type=jnp.float32)
    ex = jnp.exp(s * (1.0 / math.sqrt(hd)))
    m = jnp.dot(ex.astype(jnp.bfloat16), mht_ref[...],
                preferred_element_type=jnp.float32) * v
    ex8 = jnp.concatenate([ex] * (hid // NH), axis=1)
    slot = jax.lax.broadcasted_iota(jnp.int32, (1, hid), 1) // NH
    oh = (dstm_ref[...].astype(jnp.int32) == slot).astype(jnp.float32)
    om_ref[...] = m
    oe_ref[...] = ex8 * oh


def _edge_pass(gdst, gsrc, edge_type, e_w, dstm8, w1h, w1et, w1rf,
               b1, g, bb, w2, b2, mh, mht, hid):
    e = gdst.shape[0]
    t = 1280 if e % 1280 == 0 else 128
    grid = pl.cdiv(e, t)
    full = lambda r, c: pl.BlockSpec((r, c), lambda i: (0, 0))
    return pl.pallas_call(
        functools.partial(_edge_kernel, hid=hid),
        grid=(grid,),
        in_specs=[
            pl.BlockSpec((t, 2 * hid + 16), lambda i: (i, 0)),
            pl.BlockSpec((t, hid + 16), lambda i: (i, 0)),
            pl.BlockSpec((t, EF), lambda i: (i, 0)),
            pl.BlockSpec((t, 1), lambda i: (i, 0)),
            pl.BlockSpec((t, 1), lambda i: (i, 0)),
            full(2 * hid, 2 * hid), full(EF, 2 * hid), full(EF * NRG, 2 * hid),
            full(1, 2 * hid), full(1, 2 * hid), full(1, 2 * hid),
            full(2 * hid, 2 * hid), full(1, 2 * hid),
            full(hid, NH), full(NH, hid),
        ],
        out_specs=[pl.BlockSpec((t, hid), lambda i: (i, 0)),
                   pl.BlockSpec((t, hid), lambda i: (i, 0))],
        out_shape=[jax.ShapeDtypeStruct((e, hid), jnp.float32),
                   jax.ShapeDtypeStruct((e, hid), jnp.float32)],
    )(gdst, gsrc, edge_type, e_w, dstm8, w1h, w1et, w1rf, b1, g, bb,
      w2, b2, mh, mht)


def _sc_scatter(m, exs, dst, dst2, n, hid):
    e = m.shape[0]
    nc, ns = 2, 16
    nw = nc * ns
    per_w = e // nw
    cb = 80
    n_chunks = per_w // cb
    npad = ((n + 127) // 128) * 128
    arows = ((npad + npad // 8 + 127) // 128) * 128
    rpt = arows // ns
    zeros = jnp.zeros((arows, hid), jnp.float32)
    mesh = plsc.VectorSubcoreMesh(core_axis_name="c", subcore_axis_name="s")

    @functools.partial(
        pl.kernel, mesh=mesh,
        out_type=jax.ShapeDtypeStruct((nc, arows, hid), jnp.float32),
        scratch_types=[
            pltpu.VMEM((cb,), jnp.int32),
            pltpu.VMEM((cb,), jnp.int32),
            pltpu.VMEM((cb, hid), jnp.float32),
            pltpu.VMEM((cb, hid), jnp.float32),
            pltpu.VMEM_SHARED((arows, hid), jnp.float32),
        ],
    )
    def body(m_hbm, exs_hbm, dst_hbm, dst2_hbm, z_hbm, out_hbm, idx_v, idx2_v,
             rows_v, rows2_v, acc_sh):
        c = lax.axis_index("c")
        s = lax.axis_index("s")
        wid = c * ns + s
        base_e = wid * per_w
        pltpu.sync_copy(z_hbm.at[pl.ds(s * rpt, rpt)], acc_sh.at[pl.ds(s * rpt, rpt)])
        plsc.subcore_barrier()

        def step(i, carry):
            off = base_e + i * cb
            pltpu.sync_copy(dst_hbm.at[pl.ds(off, cb)], idx_v)
            pltpu.sync_copy(dst2_hbm.at[pl.ds(off, cb)], idx2_v)
            pltpu.sync_copy(m_hbm.at[pl.ds(off, cb)], rows_v)
            pltpu.sync_copy(exs_hbm.at[pl.ds(off, cb)], rows2_v)
            pltpu.sync_copy(rows_v, acc_sh.at[idx_v], add=True)
            pltpu.sync_copy(rows2_v, acc_sh.at[idx2_v], add=True)
            return carry

        lax.fori_loop(0, n_chunks, step, 0)
        plsc.subcore_barrier()
        pltpu.sync_copy(acc_sh.at[pl.ds(s * rpt, rpt)],
                        out_hbm.at[c].at[pl.ds(s * rpt, rpt)])

    return body(m, exs, dst, dst2, zeros)


def kernel(x, h, edge_type, edge_index, e_w, gen_flag,
           hq_w1, hq_b1, hq_ln_g, hq_ln_b, hq_w2, hq_b2,
           hk_w1, hk_b1, hk_ln_g, hk_ln_b, hk_w2, hk_b2,
           hv_w1, hv_b1, hv_ln_g, hv_ln_b, hv_w2, hv_b2):
    n, hid = h.shape
    hd = hid // NH
    src = edge_index[0].astype(jnp.int32)
    dst = edge_index[1].astype(jnp.int32)

    rs = EF + EF * NRG
    w1h = jnp.concatenate([
        jnp.concatenate([hk_w1[rs:rs + hid], hv_w1[rs:rs + hid]], axis=1),
        jnp.concatenate([hk_w1[rs + hid:], hv_w1[rs + hid:]], axis=1),
    ], axis=0)
    w1et = jnp.concatenate([hk_w1[0:EF], hv_w1[0:EF]], axis=1)
    w1rf = jnp.concatenate([hk_w1[EF:rs], hv_w1[EF:rs]], axis=1)
    b1 = jnp.concatenate([hk_b1, hv_b1]).reshape(1, -1)
    g = jnp.concatenate([hk_ln_g, hv_ln_g]).reshape(1, -1)
    bb = jnp.concatenate([hk_ln_b, hv_ln_b]).reshape(1, -1)
    zero = jnp.zeros((hid, hid), jnp.float32)
    w2 = jnp.concatenate([
        jnp.concatenate([hk_w2, zero], axis=1),
        jnp.concatenate([zero, hv_w2], axis=1),
    ], axis=0)
    b2 = jnp.concatenate([hk_b2, hv_b2]).reshape(1, -1)
    mh = (jax.lax.broadcasted_iota(jnp.int32, (hid, NH), 0) // hd ==
          jax.lax.broadcasted_iota(jnp.int32, (hid, NH), 1)).astype(jnp.bfloat16)
    mht = mh.T
    w1h = w1h.astype(jnp.bfloat16)
    w1et = w1et.astype(jnp.bfloat16)
    w1rf = w1rf.astype(jnp.bfloat16)
    w2 = w2.astype(jnp.bfloat16)

    q = _q_mlp(h, hq_w1, hq_b1, hq_ln_g, hq_ln_b, hq_w2, hq_b2)

    xpad = jnp.pad(x, ((0, 0), (0, 16 - x.shape[1])))
    tdst = jnp.concatenate([h, q, xpad], axis=1)
    tsrc = jnp.concatenate([h, xpad], axis=1)
    gdst = tdst[dst]
    gsrc = tsrc[src]

    dstm8 = (dst % 8).astype(jnp.float32).reshape(-1, 1)
    m, exs = _edge_pass(gdst, gsrc, edge_type, e_w, dstm8, w1h, w1et,
                        w1rf, b1, g, bb, w2, b2, mh, mht, hid)

    npad = ((n + 127) // 128) * 128
    dst2 = npad + dst // 8
    parts = _sc_scatter(m, exs, dst, dst2, n, hid)
    acc = parts[0] + parts[1]
    num = acc[:n, :]
    den = acc[npad:].reshape(-1, NH)[:n, :]
    out = num / (jnp.repeat(den, hd, axis=1) + 1e-16) + h
    return out
